# Initial kernel scaffold; baseline (speedup 1.0000x reference)
import functools

import jax
import jax.numpy as jnp
import numpy as np
from jax import lax
from jax.experimental import pallas as pl
from jax.experimental.pallas import tpu as pltpu
from jax.experimental.pallas import tpu_sc as plsc

F0_MIN = 32.70319566257483
F0_MAX = 2093.004522404789
N_BINS = 256
EMBED_DIM = 64

NC = 2
NS = 16
L = 16
NW = NC * NS

LUT_SIZE = 16384
BINS_PAD = 272


def _sc_embed(x_flat, table, lut, bins_pad, *, chunk):
    b = x_flat.shape[0]
    b_per_w = b // NW
    n_chunks = b_per_w // chunk
    mesh = plsc.VectorSubcoreMesh(
        core_axis_name="c", subcore_axis_name="s", num_cores=NC, num_subcores=NS
    )

    @functools.partial(
        pl.kernel,
        out_type=jax.ShapeDtypeStruct((b, EMBED_DIM), jnp.float32),
        mesh=mesh,
        scratch_types=[
            pltpu.VMEM((LUT_SIZE,), jnp.int32),
            pltpu.VMEM((BINS_PAD,), jnp.float32),
            pltpu.VMEM((chunk,), jnp.float32),
            pltpu.VMEM((chunk,), jnp.int32),
            pltpu.VMEM((chunk, EMBED_DIM), jnp.float32),
            pltpu.SemaphoreType.DMA,
        ],
    )
    def k(x_hbm, table_hbm, lut_hbm, bins_hbm, out_hbm,
          lut_v, bins_v, x_v, idx_v, rows_v, sem):
        wid = lax.axis_index("s") * NC + lax.axis_index("c")
        base = wid * b_per_w
        pltpu.sync_copy(lut_hbm, lut_v)
        pltpu.sync_copy(bins_hbm, bins_v)

        def chunk_body(ci, carry):
            lo = base + ci * chunk
            pltpu.sync_copy(x_hbm.at[pl.ds(lo, chunk)], x_v)

            def vec_body(i, carry2):
                xv = x_v[pl.ds(i * L, L)]
                bits = plsc.bitcast(xv, jnp.int32)
                t = lax.shift_right_logical(bits, 17)
                bidx = plsc.load_gather(lut_v, [t])
                c0 = plsc.load_gather(bins_v, [bidx])
                c1 = plsc.load_gather(bins_v, [bidx + 1])
                idx = (bidx + (c0 < xv).astype(jnp.int32)
                       + (c1 < xv).astype(jnp.int32))
                idx_v[pl.ds(i * L, L)] = idx
                return carry2

            lax.fori_loop(0, chunk // L, vec_body, 0)
            pltpu.async_copy(table_hbm.at[idx_v], rows_v, sem).wait()
            pltpu.sync_copy(rows_v, out_hbm.at[pl.ds(lo, chunk)])
            return carry

        lax.fori_loop(0, n_chunks, chunk_body, 0)

    return k(x_flat, table, lut, bins_pad)


def kernel(x, table):
    bins = jnp.exp(
        jnp.linspace(np.log(F0_MIN - 0.1), np.log(F0_MAX), N_BINS - 1)
    ).astype(jnp.float32)
    bins_pad = jnp.concatenate(
        [bins, jnp.full((BINS_PAD - (N_BINS - 1),), jnp.inf, jnp.float32)]
    )
    t = jnp.arange(LUT_SIZE, dtype=jnp.int32)
    bucket_min = lax.bitcast_convert_type(t << 17, jnp.float32)
    lut = jnp.searchsorted(bins, bucket_min, side="left").astype(jnp.int32)

    x_flat = x.reshape(-1)
    out = _sc_embed(x_flat, table, lut, bins_pad, chunk=512)
    return out.reshape(x.shape[0], x.shape[1], EMBED_DIM)


# speedup vs baseline: 55.3875x; 55.3875x over previous
import functools

import jax
import jax.numpy as jnp
import numpy as np
from jax import lax
from jax.experimental import pallas as pl
from jax.experimental.pallas import tpu as pltpu
from jax.experimental.pallas import tpu_sc as plsc

F0_MIN = 32.70319566257483
F0_MAX = 2093.004522404789
N_BINS = 256
EMBED_DIM = 64

NC = 2
NS = 16
L = 16
NW = NC * NS

LUT_SIZE = 16384
BINS_PAD = 272


def _sc_embed(x_flat, table, lut, bins_pad, *, chunk):
    b = x_flat.shape[0]
    b_per_w = b // NW
    n_chunks = b_per_w // chunk
    mesh = plsc.VectorSubcoreMesh(
        core_axis_name="c", subcore_axis_name="s", num_cores=NC, num_subcores=NS
    )

    @functools.partial(
        pl.kernel,
        out_type=jax.ShapeDtypeStruct((b, EMBED_DIM), jnp.float32),
        mesh=mesh,
        compiler_params=pltpu.CompilerParams(needs_layout_passes=False),
        scratch_types=[
            pltpu.VMEM((LUT_SIZE,), jnp.int32),
            pltpu.VMEM((BINS_PAD,), jnp.float32),
            pltpu.VMEM_SHARED((N_BINS, EMBED_DIM), jnp.float32),
            pltpu.VMEM((chunk,), jnp.float32),
            pltpu.VMEM((chunk,), jnp.int32),
            pltpu.VMEM((chunk, EMBED_DIM), jnp.float32),
            pltpu.SemaphoreType.DMA,
        ],
    )
    def k(x_hbm, table_hbm, lut_hbm, bins_hbm, out_hbm,
          lut_v, bins_v, table_v, x_v, idx_v, rows_v, sem):
        wid = lax.axis_index("s") * NC + lax.axis_index("c")
        base = wid * b_per_w
        pltpu.sync_copy(lut_hbm, lut_v)
        pltpu.sync_copy(bins_hbm, bins_v)
        @pl.when(lax.axis_index("s") == 0)
        def _stage_table():
            pltpu.sync_copy(table_hbm, table_v)

        plsc.subcore_barrier()

        def chunk_body(ci, carry):
            lo = base + ci * chunk
            pltpu.sync_copy(x_hbm.at[pl.ds(lo, chunk)], x_v)

            def vec_body(i, carry2):
                xv = x_v[pl.ds(i * L, L)]
                bits = lax.bitcast_convert_type(xv, jnp.int32)
                t = lax.shift_right_logical(bits, 17)
                bidx = plsc.load_gather(lut_v, [t])
                c0 = plsc.load_gather(bins_v, [bidx])
                c1 = plsc.load_gather(bins_v, [bidx + 1])
                idx = (bidx + (c0 < xv).astype(jnp.int32)
                       + (c1 < xv).astype(jnp.int32))
                idx_v[pl.ds(i * L, L)] = idx
                return carry2

            lax.fori_loop(0, chunk // L, vec_body, 0)
            pltpu.async_copy(table_v.at[idx_v], rows_v, sem).wait()
            pltpu.sync_copy(rows_v, out_hbm.at[pl.ds(lo, chunk)])
            return carry

        lax.fori_loop(0, n_chunks, chunk_body, 0)

    return k(x_flat, table, lut, bins_pad)


def kernel(x, table):
    bins = jnp.exp(
        jnp.linspace(np.log(F0_MIN - 0.1), np.log(F0_MAX), N_BINS - 1)
    ).astype(jnp.float32)
    bins_pad = jnp.concatenate(
        [bins, jnp.full((BINS_PAD - (N_BINS - 1),), jnp.inf, jnp.float32)]
    )
    t = jnp.arange(LUT_SIZE, dtype=jnp.int32)
    bucket_min = lax.bitcast_convert_type(t << 17, jnp.float32)
    lut = jnp.searchsorted(bins, bucket_min, side="left").astype(jnp.int32)

    x_flat = x.reshape(-1)
    out = _sc_embed(x_flat, table, lut, bins_pad, chunk=512)
    return out.reshape(x.shape[0], x.shape[1], EMBED_DIM)


# speedup vs baseline: 320.0275x; 5.7780x over previous
import functools

import jax
import jax.numpy as jnp
import numpy as np
from jax import lax
from jax.experimental import pallas as pl
from jax.experimental.pallas import tpu as pltpu
from jax.experimental.pallas import tpu_sc as plsc

F0_MIN = 32.70319566257483
F0_MAX = 2093.004522404789
N_BINS = 256
EMBED_DIM = 64

NC = 2
NS = 16
L = 16
NW = NC * NS

LUT_SIZE = 16384
BINS_PAD = 272


def _build_lut() -> np.ndarray:
    bins64 = np.exp(np.linspace(np.log(F0_MIN - 0.1), np.log(F0_MAX), N_BINS - 1))
    bins32 = bins64.astype(np.float32)
    bins_hi = bins32 + 64.0 * np.spacing(bins32)
    t = np.arange(LUT_SIZE, dtype=np.int64)
    bucket_min = (t << 17).astype(np.int32).view(np.float32)
    return np.searchsorted(bins_hi, bucket_min, side="left").astype(np.int32)


_LUT_NP = _build_lut()


def _sc_bucketize(x_flat, lut, bins_pad, *, chunk):
    b = x_flat.shape[0]
    b_per_w = b // NW
    n_chunks = b_per_w // chunk
    mesh = plsc.VectorSubcoreMesh(
        core_axis_name="c", subcore_axis_name="s", num_cores=NC, num_subcores=NS
    )

    @functools.partial(
        pl.kernel,
        out_type=jax.ShapeDtypeStruct((b,), jnp.int32),
        mesh=mesh,
        compiler_params=pltpu.CompilerParams(needs_layout_passes=False),
        scratch_types=[
            pltpu.VMEM((LUT_SIZE,), jnp.int32),
            pltpu.VMEM((BINS_PAD,), jnp.float32),
            pltpu.VMEM((chunk,), jnp.float32),
            pltpu.VMEM((chunk,), jnp.int32),
        ],
    )
    def k(x_hbm, lut_hbm, bins_hbm, out_hbm, lut_v, bins_v, x_v, idx_v):
        wid = lax.axis_index("s") * NC + lax.axis_index("c")
        base = wid * b_per_w
        pltpu.sync_copy(lut_hbm, lut_v)
        pltpu.sync_copy(bins_hbm, bins_v)

        def chunk_body(ci, carry):
            lo = base + ci * chunk
            pltpu.sync_copy(x_hbm.at[pl.ds(lo, chunk)], x_v)

            def vec_body(i, carry2):
                xv = x_v[pl.ds(i * L, L)]
                bits = lax.bitcast_convert_type(xv, jnp.int32)
                t = lax.shift_right_logical(bits, 17)
                bidx = plsc.load_gather(lut_v, [t])
                c0 = plsc.load_gather(bins_v, [bidx])
                c1 = plsc.load_gather(bins_v, [bidx + 1])
                c2 = plsc.load_gather(bins_v, [bidx + 2])
                idx = (bidx + (c0 < xv).astype(jnp.int32)
                       + (c1 < xv).astype(jnp.int32)
                       + (c2 < xv).astype(jnp.int32))
                idx_v[pl.ds(i * L, L)] = idx
                return carry2

            lax.fori_loop(0, chunk // L, vec_body, 0)
            pltpu.sync_copy(idx_v, out_hbm.at[pl.ds(lo, chunk)])
            return carry

        lax.fori_loop(0, n_chunks, chunk_body, 0)

    return k(x_flat, lut, bins_pad)


def _tc_embed(idx_t3, table_t_hi, table_t_lo, *, seq, batch):
    def body(idx_ref, thi_ref, tlo_ref, out_ref):
        idx_row = idx_ref[0]
        ks = lax.broadcasted_iota(jnp.int32, (N_BINS, batch), 0)
        oh = (ks == idx_row).astype(jnp.bfloat16)
        hi = lax.dot_general(thi_ref[...], oh, (((1,), (0,)), ((), ())),
                             preferred_element_type=jnp.float32)
        lo = lax.dot_general(tlo_ref[...], oh, (((1,), (0,)), ((), ())),
                             preferred_element_type=jnp.float32)
        out_ref[0] = hi + lo

    return pl.pallas_call(
        body,
        grid=(seq,),
        in_specs=[
            pl.BlockSpec((1, 1, batch), lambda i: (i, 0, 0)),
            pl.BlockSpec((EMBED_DIM, N_BINS), lambda i: (0, 0)),
            pl.BlockSpec((EMBED_DIM, N_BINS), lambda i: (0, 0)),
        ],
        out_specs=pl.BlockSpec((1, EMBED_DIM, batch), lambda i: (i, 0, 0)),
        out_shape=jax.ShapeDtypeStruct((seq, EMBED_DIM, batch), jnp.float32),
    )(idx_t3, table_t_hi, table_t_lo)


def kernel(x, table):
    batch, seq = x.shape
    bins = jnp.exp(
        jnp.linspace(np.log(F0_MIN - 0.1), np.log(F0_MAX), N_BINS - 1)
    ).astype(jnp.float32)
    bins_pad = jnp.concatenate(
        [bins, jnp.full((BINS_PAD - (N_BINS - 1),), jnp.inf, jnp.float32)]
    )
    lut = jnp.asarray(_LUT_NP)

    xt_flat = x.T.reshape(-1)
    idx_flat = _sc_bucketize(xt_flat, lut, bins_pad, chunk=512)
    idx_t3 = idx_flat.reshape(seq, 1, batch)

    t_hi = table.astype(jnp.bfloat16)
    t_lo = (table - t_hi.astype(jnp.float32)).astype(jnp.bfloat16)
    out3 = _tc_embed(idx_t3, t_hi.T, t_lo.T, seq=seq, batch=batch)
    return jnp.transpose(out3, (2, 0, 1))
